# baseline (speedup 1.0000x reference)
import functools
import math

import jax
import jax.numpy as jnp
from jax import lax
from jax.experimental import pallas as pl
from jax.experimental.pallas import tpu as pltpu
from jax.experimental.pallas import tpu_sc as plsc

VOCAB = 100000
D_MODEL = 128
BATCH = 1024
SEQ = 200
SCALE = math.sqrt(float(D_MODEL))

NUM_CORES = 2
NUM_SUBCORES = 16
NW = NUM_CORES * NUM_SUBCORES
TOTAL_ROWS = BATCH * SEQ
ROWS_PER_W = TOTAL_ROWS // NW
CHUNK = 100
NCHUNK = ROWS_PER_W // CHUNK
NBUF = 4


def _sc_body(x_hbm, table_hbm, out_hbm, idx_v, rin, rout, gsem, ssem):
  cid = lax.axis_index("c")
  sid = lax.axis_index("s")
  wid = sid * NUM_CORES + cid

  pltpu.sync_copy(x_hbm.at[wid], idx_v)

  def gather(c, b):
    return pltpu.make_async_copy(table_hbm.at[idx_v.at[c]], rin[b], gsem[b])

  def scatter(c, b):
    return pltpu.make_async_copy(rout[b], out_hbm.at[wid, c], ssem[b])

  for b in range(NBUF):
    gather(b, b).start()

  def group(g0, _):
    g = g0 * NBUF
    for b in range(NBUF):
      c = g + b
      gather(c, b).wait()

      @pl.when(c >= NBUF)
      def _(b=b, c=c):
        scatter(c - NBUF, b).wait()

      def scale_row(i, _, b=b):
        for j in range(D_MODEL // 16):
          sl = pl.ds(j * 16, 16)
          rout[b][i, sl] = rin[b][i, sl] * SCALE
        return 0

      lax.fori_loop(0, CHUNK, scale_row, 0)

      scatter(c, b).start()

      @pl.when(c + NBUF < NCHUNK)
      def _(b=b, c=c):
        gather(c + NBUF, b).start()
    return 0

  lax.fori_loop(0, NCHUNK // NBUF, group, 0)

  for b in range(NBUF):
    scatter(NCHUNK - NBUF + b, b).wait()


@jax.jit
def _run(x_flat, table):
  mesh = plsc.VectorSubcoreMesh(core_axis_name="c", subcore_axis_name="s")
  f = pl.kernel(
      _sc_body,
      out_type=jax.ShapeDtypeStruct((NW, NCHUNK, CHUNK, D_MODEL), jnp.float32),
      mesh=mesh,
      scratch_types=[
          pltpu.VMEM((NCHUNK, CHUNK), jnp.int32),
          [pltpu.VMEM((CHUNK, D_MODEL), jnp.float32) for _ in range(NBUF)],
          [pltpu.VMEM((CHUNK, D_MODEL), jnp.float32) for _ in range(NBUF)],
          [pltpu.SemaphoreType.DMA for _ in range(NBUF)],
          [pltpu.SemaphoreType.DMA for _ in range(NBUF)],
      ],
  )
  return f(x_flat, table)


def kernel(x, table):
  x_flat = x.reshape(NW, NCHUNK, CHUNK)
  out = _run(x_flat, table)
  return out.reshape(BATCH, SEQ, D_MODEL)


# speedup vs baseline: 2.0233x; 2.0233x over previous
import functools
import math

import jax
import jax.numpy as jnp
from jax import lax
from jax.experimental import pallas as pl
from jax.experimental.pallas import tpu as pltpu
from jax.experimental.pallas import tpu_sc as plsc

VOCAB = 100000
D_MODEL = 128
BATCH = 1024
SEQ = 200
SCALE = math.sqrt(float(D_MODEL))

NUM_CORES = 2
NUM_SUBCORES = 16
NW = NUM_CORES * NUM_SUBCORES
TOTAL_ROWS = BATCH * SEQ
ROWS_PER_W = TOTAL_ROWS // NW
CHUNK = 128
NCHUNK = ROWS_PER_W // CHUNK
NBUF = 5
LOOKAHEAD = 3


def _sc_body(x_hbm, table_hbm, out_hbm, idx_v, bufs, gsem, ssem):
  cid = lax.axis_index("c")
  sid = lax.axis_index("s")
  wid = sid * NUM_CORES + cid

  pltpu.sync_copy(x_hbm.at[wid], idx_v)

  def gather(c, b):
    return pltpu.make_async_copy(table_hbm.at[idx_v.at[c]], bufs[b], gsem[b])

  def scatter(c, b):
    return pltpu.make_async_copy(bufs[b], out_hbm.at[wid, c], ssem[b])

  for b in range(LOOKAHEAD):
    gather(b, b).start()

  def group(g0, _):
    g = g0 * NBUF
    for b in range(NBUF):
      c = g + b
      nb = (b + LOOKAHEAD) % NBUF

      @pl.when(c + LOOKAHEAD < NCHUNK)
      def _(c=c, nb=nb):
        @pl.when(c + LOOKAHEAD >= NBUF)
        def _():
          scatter(c + LOOKAHEAD - NBUF, nb).wait()
        gather(c + LOOKAHEAD, nb).start()

      gather(c, b).wait()

      def scale_row(i, _, b=b):
        for j in range(D_MODEL // 16):
          sl = pl.ds(j * 16, 16)
          bufs[b][i, sl] = bufs[b][i, sl] * SCALE
        return 0

      lax.fori_loop(0, CHUNK, scale_row, 0)

      scatter(c, b).start()
    return 0

  lax.fori_loop(0, NCHUNK // NBUF, group, 0)

  for b in range(NBUF):
    scatter(NCHUNK - NBUF + b, (NCHUNK - NBUF + b) % NBUF).wait()


@jax.jit
def _run(x_flat, table):
  mesh = plsc.VectorSubcoreMesh(core_axis_name="c", subcore_axis_name="s")
  f = pl.kernel(
      _sc_body,
      out_type=jax.ShapeDtypeStruct((NW, NCHUNK, CHUNK, D_MODEL), jnp.float32),
      mesh=mesh,
      scratch_types=[
          pltpu.VMEM((NCHUNK, CHUNK), jnp.int32),
          [pltpu.VMEM((CHUNK, D_MODEL), jnp.float32) for _ in range(NBUF)],
          [pltpu.SemaphoreType.DMA for _ in range(NBUF)],
          [pltpu.SemaphoreType.DMA for _ in range(NBUF)],
      ],
  )
  return f(x_flat, table)


def kernel(x, table):
  x_flat = x.reshape(NW, NCHUNK, CHUNK)
  out = _run(x_flat, table)
  return out.reshape(BATCH, SEQ, D_MODEL)


# speedup vs baseline: 2.0240x; 1.0003x over previous
import functools
import math

import jax
import jax.numpy as jnp
from jax import lax
from jax.experimental import pallas as pl
from jax.experimental.pallas import tpu as pltpu
from jax.experimental.pallas import tpu_sc as plsc

VOCAB = 100000
D_MODEL = 128
BATCH = 1024
SEQ = 200
SCALE = math.sqrt(float(D_MODEL))

NUM_CORES = 2
NUM_SUBCORES = 16
NW = NUM_CORES * NUM_SUBCORES
TOTAL_ROWS = BATCH * SEQ
ROWS_PER_W = TOTAL_ROWS // NW
CHUNK = 64
NCHUNK = ROWS_PER_W // CHUNK
NBUF = 10
LOOKAHEAD = 5


def _sc_body(x_hbm, table_hbm, out_hbm, idx_v, bufs, gsem, ssem):
  cid = lax.axis_index("c")
  sid = lax.axis_index("s")
  wid = sid * NUM_CORES + cid

  pltpu.sync_copy(x_hbm.at[wid], idx_v)

  def gather(c, b):
    return pltpu.make_async_copy(table_hbm.at[idx_v.at[c]], bufs[b], gsem[b])

  def scatter(c, b):
    return pltpu.make_async_copy(bufs[b], out_hbm.at[wid, c], ssem[b])

  for b in range(LOOKAHEAD):
    gather(b, b).start()

  def group(g0, _):
    g = g0 * NBUF
    for b in range(NBUF):
      c = g + b
      nb = (b + LOOKAHEAD) % NBUF

      @pl.when(c + LOOKAHEAD < NCHUNK)
      def _(c=c, nb=nb):
        @pl.when(c + LOOKAHEAD >= NBUF)
        def _():
          scatter(c + LOOKAHEAD - NBUF, nb).wait()
        gather(c + LOOKAHEAD, nb).start()

      gather(c, b).wait()

      def scale_row(i, _, b=b):
        for j in range(D_MODEL // 16):
          sl = pl.ds(j * 16, 16)
          bufs[b][i, sl] = bufs[b][i, sl] * SCALE
        return 0

      lax.fori_loop(0, CHUNK, scale_row, 0)

      scatter(c, b).start()
    return 0

  lax.fori_loop(0, NCHUNK // NBUF, group, 0)

  for b in range(NBUF):
    scatter(NCHUNK - NBUF + b, (NCHUNK - NBUF + b) % NBUF).wait()


@jax.jit
def _run(x_flat, table):
  mesh = plsc.VectorSubcoreMesh(core_axis_name="c", subcore_axis_name="s")
  f = pl.kernel(
      _sc_body,
      out_type=jax.ShapeDtypeStruct((NW, NCHUNK, CHUNK, D_MODEL), jnp.float32),
      mesh=mesh,
      scratch_types=[
          pltpu.VMEM((NCHUNK, CHUNK), jnp.int32),
          [pltpu.VMEM((CHUNK, D_MODEL), jnp.float32) for _ in range(NBUF)],
          [pltpu.SemaphoreType.DMA for _ in range(NBUF)],
          [pltpu.SemaphoreType.DMA for _ in range(NBUF)],
      ],
  )
  return f(x_flat, table)


def kernel(x, table):
  x_flat = x.reshape(NW, NCHUNK, CHUNK)
  out = _run(x_flat, table)
  return out.reshape(BATCH, SEQ, D_MODEL)
